# baseline (speedup 1.0000x reference)
import jax
import jax.numpy as jnp
from jax import lax
from jax.experimental import pallas as pl
from jax.experimental.pallas import tpu as pltpu
from jax.experimental.pallas import tpu_sc as plsc

B = 16384
EMB = 32
MLP = 128

_info = plsc.get_sparse_core_info()
_NC, _NS = _info.num_cores, _info.num_subcores
_NW = _NC * _NS
_RPW = B // _NW
_CH = 128
_NCH = _RPW // _CH


def _sc_mlp_body(user_h, item_h, um_h, im_h, oum_h, oim_h,
                 idx_u, idx_i, buf_m, sem_m):
    c = lax.axis_index("c")
    s = lax.axis_index("s")
    wid = s * _NC + c
    base = wid * _RPW
    pltpu.sync_copy(user_h.at[pl.ds(wid * _NCH, _NCH)], idx_u)
    pltpu.sync_copy(item_h.at[pl.ds(wid * _NCH, _NCH)], idx_i)
    um_cps = [pltpu.async_copy(um_h.at[idx_u.at[j]],
                               buf_m.at[pl.ds(j * _CH, _CH)], sem_m)
              for j in range(_NCH)]
    for cp in um_cps:
        cp.wait()
    pltpu.sync_copy(buf_m, oum_h.at[pl.ds(base, _RPW)])
    im_cps = [pltpu.async_copy(im_h.at[idx_i.at[j]],
                               buf_m.at[pl.ds(j * _CH, _CH)], sem_m)
              for j in range(_NCH)]
    for cp in im_cps:
        cp.wait()
    pltpu.sync_copy(buf_m, oim_h.at[pl.ds(base, _RPW)])


def _sc_gmf_body(idx2d_h, tab_h, out_h,
                 idx, buf_ga, buf_gb, sem_ga, sem_gb):
    c = lax.axis_index("c")
    s = lax.axis_index("s")
    wid = s * _NC + c
    base = wid * _RPW
    pltpu.sync_copy(idx2d_h.at[pl.ds(wid * _NCH, _NCH)], idx)

    def fire(j, buf, sem):
        def body(g, _):
            vec = idx[j, pl.ds(g * 16, 16)]
            for l in range(16):
                pltpu.async_copy(tab_h.at[pl.ds(vec[l], 1)],
                                 buf.at[pl.ds(g * 16 + l, 1)], sem)
            return 0
        lax.fori_loop(0, _CH // 16, body, 0)

    def drain(j, buf, sem):
        pltpu.make_async_copy(tab_h.at[pl.ds(0, _CH)], buf, sem).wait()
        pltpu.sync_copy(buf, out_h.at[pl.ds(base + j * _CH, _CH)])

    bufs = [(buf_ga, sem_ga), (buf_gb, sem_gb)]
    fire(0, *bufs[0])
    for k in range(1, _NCH):
        fire(k, *bufs[k % 2])
        drain(k - 1, *bufs[(k - 1) % 2])
    drain(_NCH - 1, *bufs[(_NCH - 1) % 2])


def _sc_gather_mlp(user2d, item2d, um, im):
    mesh = plsc.VectorSubcoreMesh(core_axis_name="c", subcore_axis_name="s")
    f32 = jnp.float32
    out_type = [
        jax.ShapeDtypeStruct((B, MLP), f32),
        jax.ShapeDtypeStruct((B, MLP), f32),
    ]
    scratch = [
        pltpu.VMEM((_NCH, _CH), jnp.int32),
        pltpu.VMEM((_NCH, _CH), jnp.int32),
        pltpu.VMEM((_RPW, MLP), f32),
        pltpu.SemaphoreType.DMA,
    ]
    return pl.kernel(
        _sc_mlp_body, mesh=mesh, out_type=out_type, scratch_types=scratch,
    )(user2d, item2d, um, im)


def _sc_gather_gmf(idx2d, tab, sc_format):
    mesh = plsc.VectorSubcoreMesh(core_axis_name="c", subcore_axis_name="s")
    f32 = jnp.float32
    scratch = [
        pltpu.VMEM((_NCH, _CH), jnp.int32),
        pltpu.VMEM((_CH, EMB), f32),
        pltpu.VMEM((_CH, EMB), f32),
        pltpu.SemaphoreType.DMA,
        pltpu.SemaphoreType.DMA,
    ]
    params = pltpu.CompilerParams(use_tc_tiling_on_sc=not sc_format)
    return pl.kernel(
        _sc_gmf_body, mesh=mesh,
        out_type=jax.ShapeDtypeStruct((B, EMB), f32),
        scratch_types=scratch, compiler_params=params,
    )(idx2d, tab)


_TB = 1024


def _tc_trunk_body(uem, iem, w1a, w1b, b1, w2, b2, w3, b3, out):
    f32 = jnp.float32
    h = jnp.maximum(
        jnp.dot(uem[...], w1a[...], preferred_element_type=f32)
        + jnp.dot(iem[...], w1b[...], preferred_element_type=f32)
        + b1[...], 0.0)
    h = jnp.maximum(jnp.dot(h, w2[...], preferred_element_type=f32)
                    + b2[...], 0.0)
    h = jnp.maximum(jnp.dot(h, w3[...], preferred_element_type=f32)
                    + b3[...], 0.0)
    out[...] = h


def _tc_combine_body(ueg, ieg, h, wpg, wph, bp, out):
    f32 = jnp.float32
    g = ueg[...] * ieg[...]
    out[...] = (jnp.dot(g, wpg[...], preferred_element_type=f32)
                + jnp.dot(h[...], wph[...], preferred_element_type=f32)
                + bp[...])


def _tc_trunk(uem, iem, w1a, w1b, b1, w2, b2, w3, b3):
    def rows(d):
        return pl.BlockSpec((_TB, d), lambda i: (i, 0))

    def full2(a, b):
        return pl.BlockSpec((a, b), lambda i: (0, 0))

    def full1(a):
        return pl.BlockSpec((a,), lambda i: (0,))

    return pl.pallas_call(
        _tc_trunk_body,
        grid=(B // _TB,),
        in_specs=[
            rows(MLP), rows(MLP),
            full2(MLP, 128), full2(MLP, 128), full1(128),
            full2(128, 64), full1(64),
            full2(64, 32), full1(32),
        ],
        out_specs=pl.BlockSpec((_TB, 32), lambda i: (i, 0)),
        out_shape=jax.ShapeDtypeStruct((B, 32), jnp.float32),
    )(uem, iem, w1a, w1b, b1, w2, b2, w3, b3)


def _tc_combine(ueg, ieg, h, wpg, wph, bp2):
    def rows(d):
        return pl.BlockSpec((_TB, d), lambda i: (i, 0))

    def full2(a, b):
        return pl.BlockSpec((a, b), lambda i: (0, 0))

    return pl.pallas_call(
        _tc_combine_body,
        grid=(B // _TB,),
        in_specs=[
            rows(EMB), rows(EMB), rows(32),
            full2(EMB, 1), full2(32, 1), full2(1, 1),
        ],
        out_specs=pl.BlockSpec((_TB, 1), lambda i: (i, 0)),
        out_shape=jax.ShapeDtypeStruct((B, 1), jnp.float32),
    )(ueg, ieg, h, wpg, wph, bp2)


def kernel(user, item, user_gmf, item_gmf, user_mlp, item_mlp,
           W1, b1, W2, b2, W3, b3, Wp, bp):
    user2d = user.astype(jnp.int32).reshape(_NW * _NCH, _CH)
    item2d = item.astype(jnp.int32).reshape(_NW * _NCH, _CH)
    uem, iem = _sc_gather_mlp(user2d, item2d, user_mlp, item_mlp)
    ieg = _sc_gather_gmf(item2d, item_gmf, sc_format=True)
    ueg = _sc_gather_gmf(user2d, user_gmf, sc_format=False)
    W1a, W1b = W1[:MLP], W1[MLP:]
    Wpg, Wph = Wp[:EMB], Wp[EMB:]
    h = _tc_trunk(uem, iem, W1a, W1b, b1, W2, b2, W3, b3)
    out = _tc_combine(ueg, ieg, h, Wpg, Wph, bp.reshape(1, 1))
    return out.reshape(B)


# speedup vs baseline: 1.3316x; 1.3316x over previous
import jax
import jax.numpy as jnp
from jax import lax
from jax.experimental import pallas as pl
from jax.experimental.pallas import tpu as pltpu
from jax.experimental.pallas import tpu_sc as plsc

B = 16384
EMB = 32
MLP = 128

_info = plsc.get_sparse_core_info()
_NC, _NS = _info.num_cores, _info.num_subcores
_NW = _NC * _NS
_RPW = B // _NW
_CH = 128
_NCH = _RPW // _CH


def _sc_mlp_body(user_h, item_h, um_h, im_h, oum_h, oim_h,
                 idx_u, idx_i, buf_m, sem_m):
    c = lax.axis_index("c")
    s = lax.axis_index("s")
    wid = s * _NC + c
    base = wid * _RPW
    pltpu.sync_copy(user_h.at[pl.ds(wid * _NCH, _NCH)], idx_u)
    pltpu.sync_copy(item_h.at[pl.ds(wid * _NCH, _NCH)], idx_i)
    um_cps = [pltpu.async_copy(um_h.at[idx_u.at[j]],
                               buf_m.at[pl.ds(j * _CH, _CH)], sem_m)
              for j in range(_NCH)]
    for cp in um_cps:
        cp.wait()
    pltpu.sync_copy(buf_m, oum_h.at[pl.ds(base, _RPW)])
    im_cps = [pltpu.async_copy(im_h.at[idx_i.at[j]],
                               buf_m.at[pl.ds(j * _CH, _CH)], sem_m)
              for j in range(_NCH)]
    for cp in im_cps:
        cp.wait()
    pltpu.sync_copy(buf_m, oim_h.at[pl.ds(base, _RPW)])


def _sc_gmf_body(idx2d_h, tab_h, out_h,
                 idx, buf_ga, buf_gb, sem_ga, sem_gb):
    c = lax.axis_index("c")
    s = lax.axis_index("s")
    wid = s * _NC + c
    base = wid * _RPW
    pltpu.sync_copy(idx2d_h.at[pl.ds(wid * _NCH, _NCH)], idx)

    def fire(j, buf, sem):
        def body(g, _):
            vec = idx[j, pl.ds(g * 16, 16)]
            for l in range(16):
                pltpu.async_copy(tab_h.at[pl.ds(vec[l], 1)],
                                 buf.at[pl.ds(g * 16 + l, 1)], sem)
            return 0
        lax.fori_loop(0, _CH // 16, body, 0)

    def drain(j, buf, sem):
        pltpu.make_async_copy(tab_h.at[pl.ds(0, _CH)], buf, sem).wait()
        pltpu.sync_copy(buf, out_h.at[pl.ds(base + j * _CH, _CH)])

    bufs = [(buf_ga, sem_ga), (buf_gb, sem_gb)]
    fire(0, *bufs[0])
    for k in range(1, _NCH):
        fire(k, *bufs[k % 2])
        drain(k - 1, *bufs[(k - 1) % 2])
    drain(_NCH - 1, *bufs[(_NCH - 1) % 2])


def _sc_gather_mlp(user2d, item2d, um, im):
    mesh = plsc.VectorSubcoreMesh(core_axis_name="c", subcore_axis_name="s")
    f32 = jnp.float32
    out_type = [
        jax.ShapeDtypeStruct((B, MLP), f32),
        jax.ShapeDtypeStruct((B, MLP), f32),
    ]
    scratch = [
        pltpu.VMEM((_NCH, _CH), jnp.int32),
        pltpu.VMEM((_NCH, _CH), jnp.int32),
        pltpu.VMEM((_RPW, MLP), f32),
        pltpu.SemaphoreType.DMA,
    ]
    return pl.kernel(
        _sc_mlp_body, mesh=mesh, out_type=out_type, scratch_types=scratch,
    )(user2d, item2d, um, im)


def _sc_gather_gmf(idx2d, tab):
    mesh = plsc.VectorSubcoreMesh(core_axis_name="c", subcore_axis_name="s")
    f32 = jnp.float32
    scratch = [
        pltpu.VMEM((_NCH, _CH), jnp.int32),
        pltpu.VMEM((_CH, EMB), f32),
        pltpu.VMEM((_CH, EMB), f32),
        pltpu.SemaphoreType.DMA,
        pltpu.SemaphoreType.DMA,
    ]
    return pl.kernel(
        _sc_gmf_body, mesh=mesh,
        out_type=jax.ShapeDtypeStruct((B, EMB), f32),
        scratch_types=scratch,
    )(idx2d, tab)


_TB = 1024


def _tc_trunk_body(uem, iem, w1a, w1b, b1, w2, b2, w3, b3, out):
    f32 = jnp.float32
    h = jnp.maximum(
        jnp.dot(uem[...], w1a[...], preferred_element_type=f32)
        + jnp.dot(iem[...], w1b[...], preferred_element_type=f32)
        + b1[...], 0.0)
    h = jnp.maximum(jnp.dot(h, w2[...], preferred_element_type=f32)
                    + b2[...], 0.0)
    h = jnp.maximum(jnp.dot(h, w3[...], preferred_element_type=f32)
                    + b3[...], 0.0)
    out[...] = h


def _tc_combine_body(ueg, ieg, h, wpg, wph, bp, out):
    f32 = jnp.float32
    g = ueg[...] * ieg[...]
    pred = (jnp.dot(g, wpg[...], preferred_element_type=f32)
            + jnp.dot(h[...], wph[...], preferred_element_type=f32)
            + bp[...])
    out[...] = pred[:, 0]


def _tc_trunk(uem, iem, w1a, w1b, b1, w2, b2, w3, b3):
    def rows(d):
        return pl.BlockSpec((_TB, d), lambda i: (i, 0))

    def full2(a, b):
        return pl.BlockSpec((a, b), lambda i: (0, 0))

    def full1(a):
        return pl.BlockSpec((a,), lambda i: (0,))

    return pl.pallas_call(
        _tc_trunk_body,
        grid=(B // _TB,),
        in_specs=[
            rows(MLP), rows(MLP),
            full2(MLP, 128), full2(MLP, 128), full1(128),
            full2(128, 64), full1(64),
            full2(64, 32), full1(32),
        ],
        out_specs=pl.BlockSpec((_TB, 32), lambda i: (i, 0)),
        out_shape=jax.ShapeDtypeStruct((B, 32), jnp.float32),
    )(uem, iem, w1a, w1b, b1, w2, b2, w3, b3)


def _tc_combine(ueg, ieg, h, wpg, wph, bp2):
    def rows(d):
        return pl.BlockSpec((_TB, d), lambda i: (i, 0))

    def full2(a, b):
        return pl.BlockSpec((a, b), lambda i: (0, 0))

    return pl.pallas_call(
        _tc_combine_body,
        grid=(B // _TB,),
        in_specs=[
            rows(EMB), rows(EMB), rows(32),
            full2(EMB, 1), full2(32, 1), full2(1, 1),
        ],
        out_specs=pl.BlockSpec((_TB,), lambda i: (i,)),
        out_shape=jax.ShapeDtypeStruct((B,), jnp.float32),
    )(ueg, ieg, h, wpg, wph, bp2)


def kernel(user, item, user_gmf, item_gmf, user_mlp, item_mlp,
           W1, b1, W2, b2, W3, b3, Wp, bp):
    user2d = user.astype(jnp.int32).reshape(_NW * _NCH, _CH)
    item2d = item.astype(jnp.int32).reshape(_NW * _NCH, _CH)
    uem, iem = _sc_gather_mlp(user2d, item2d, user_mlp, item_mlp)
    ueg = _sc_gather_gmf(user2d, user_gmf)
    ieg = _sc_gather_gmf(item2d, item_gmf)
    W1a, W1b = W1[:MLP], W1[MLP:]
    Wpg, Wph = Wp[:EMB], Wp[EMB:]
    h = _tc_trunk(uem, iem, W1a, W1b, b1, W2, b2, W3, b3)
    return _tc_combine(ueg, ieg, h, Wpg, Wph, bp.reshape(1, 1))


# speedup vs baseline: 1.3456x; 1.0105x over previous
import jax
import jax.numpy as jnp
from jax import lax
from jax.experimental import pallas as pl
from jax.experimental.pallas import tpu as pltpu
from jax.experimental.pallas import tpu_sc as plsc

B = 16384
EMB = 32
MLP = 128

_info = plsc.get_sparse_core_info()
_NC, _NS = _info.num_cores, _info.num_subcores
_NW = _NC * _NS
_RPW = B // _NW
_CH = 128
_NCH = _RPW // _CH


def _sc_mlp_body(user_h, item_h, um_h, im_h, oum_h, oim_h,
                 idx_u, idx_i, buf_m, sem_m):
    c = lax.axis_index("c")
    s = lax.axis_index("s")
    wid = s * _NC + c
    base = wid * _RPW
    pltpu.sync_copy(user_h.at[pl.ds(wid * _NCH, _NCH)], idx_u)
    pltpu.sync_copy(item_h.at[pl.ds(wid * _NCH, _NCH)], idx_i)
    um_cps = [pltpu.async_copy(um_h.at[idx_u.at[j]],
                               buf_m.at[pl.ds(j * _CH, _CH)], sem_m)
              for j in range(_NCH)]
    for cp in um_cps:
        cp.wait()
    pltpu.sync_copy(buf_m, oum_h.at[pl.ds(base, _RPW)])
    im_cps = [pltpu.async_copy(im_h.at[idx_i.at[j]],
                               buf_m.at[pl.ds(j * _CH, _CH)], sem_m)
              for j in range(_NCH)]
    for cp in im_cps:
        cp.wait()
    pltpu.sync_copy(buf_m, oim_h.at[pl.ds(base, _RPW)])


def _sc_gmf_body(idx2d_h, tab_h, out_h,
                 idx, buf_ga, buf_gb, sem_ga, sem_gb):
    c = lax.axis_index("c")
    s = lax.axis_index("s")
    wid = s * _NC + c
    base = wid * _RPW
    pltpu.sync_copy(idx2d_h.at[pl.ds(wid * _NCH, _NCH)], idx)

    def fire(j, buf, sem):
        def body(g, _):
            vec = idx[j, pl.ds(g * 16, 16)]
            for l in range(16):
                pltpu.async_copy(tab_h.at[pl.ds(vec[l], 1)],
                                 buf.at[pl.ds(g * 16 + l, 1)], sem)
            return 0
        lax.fori_loop(0, _CH // 16, body, 0)

    def drain(j, buf, sem):
        pltpu.make_async_copy(tab_h.at[pl.ds(0, _CH)], buf, sem).wait()
        pltpu.sync_copy(buf, out_h.at[pl.ds(base + j * _CH, _CH)])

    bufs = [(buf_ga, sem_ga), (buf_gb, sem_gb)]
    fire(0, *bufs[0])
    for k in range(1, _NCH):
        fire(k, *bufs[k % 2])
        drain(k - 1, *bufs[(k - 1) % 2])
    drain(_NCH - 1, *bufs[(_NCH - 1) % 2])


def _sc_gather_mlp(user2d, item2d, um, im):
    mesh = plsc.VectorSubcoreMesh(core_axis_name="c", subcore_axis_name="s")
    f32 = jnp.float32
    out_type = [
        jax.ShapeDtypeStruct((B, MLP), f32),
        jax.ShapeDtypeStruct((B, MLP), f32),
    ]
    scratch = [
        pltpu.VMEM((_NCH, _CH), jnp.int32),
        pltpu.VMEM((_NCH, _CH), jnp.int32),
        pltpu.VMEM((_RPW, MLP), f32),
        pltpu.SemaphoreType.DMA,
    ]
    return pl.kernel(
        _sc_mlp_body, mesh=mesh, out_type=out_type, scratch_types=scratch,
    )(user2d, item2d, um, im)


def _sc_gather_gmf(idx2d, tab):
    mesh = plsc.VectorSubcoreMesh(core_axis_name="c", subcore_axis_name="s")
    f32 = jnp.float32
    scratch = [
        pltpu.VMEM((_NCH, _CH), jnp.int32),
        pltpu.VMEM((_CH, EMB), f32),
        pltpu.VMEM((_CH, EMB), f32),
        pltpu.SemaphoreType.DMA,
        pltpu.SemaphoreType.DMA,
    ]
    return pl.kernel(
        _sc_gmf_body, mesh=mesh,
        out_type=jax.ShapeDtypeStruct((B, EMB), f32),
        scratch_types=scratch,
    )(idx2d, tab)


_TB = 2048


def _tc_body(ueg, ieg, uem, iem, w1a, w1b, b1, w2, b2, w3, b3,
             wpg, wph, bp, out):
    f32 = jnp.float32
    g = ueg[...] * ieg[...]
    h = jnp.maximum(
        jnp.dot(uem[...], w1a[...], preferred_element_type=f32)
        + jnp.dot(iem[...], w1b[...], preferred_element_type=f32)
        + b1[...], 0.0)
    h = jnp.maximum(jnp.dot(h, w2[...], preferred_element_type=f32)
                    + b2[...], 0.0)
    h = jnp.maximum(jnp.dot(h, w3[...], preferred_element_type=f32)
                    + b3[...], 0.0)
    pred = (jnp.dot(g, wpg[...], preferred_element_type=f32)
            + jnp.dot(h, wph[...], preferred_element_type=f32)
            + bp[...])
    out[...] = pred[:, 0]


def _tc_mlp(ueg, ieg, uem, iem, w1a, w1b, b1, w2, b2, w3, b3, wpg, wph, bp2):
    def rows(d):
        return pl.BlockSpec((_TB, d), lambda i: (i, 0))

    def full2(a, b):
        return pl.BlockSpec((a, b), lambda i: (0, 0))

    def full1(a):
        return pl.BlockSpec((a,), lambda i: (0,))

    return pl.pallas_call(
        _tc_body,
        grid=(B // _TB,),
        in_specs=[
            rows(EMB), rows(EMB), rows(MLP), rows(MLP),
            full2(MLP, 128), full2(MLP, 128), full1(128),
            full2(128, 64), full1(64),
            full2(64, 32), full1(32),
            full2(EMB, 1), full2(32, 1), full2(1, 1),
        ],
        out_specs=pl.BlockSpec((_TB,), lambda i: (i,)),
        out_shape=jax.ShapeDtypeStruct((B,), jnp.float32),
    )(ueg, ieg, uem, iem, w1a, w1b, b1, w2, b2, w3, b3, wpg, wph, bp2)


def kernel(user, item, user_gmf, item_gmf, user_mlp, item_mlp,
           W1, b1, W2, b2, W3, b3, Wp, bp):
    user2d = user.astype(jnp.int32).reshape(_NW * _NCH, _CH)
    item2d = item.astype(jnp.int32).reshape(_NW * _NCH, _CH)
    uem, iem = _sc_gather_mlp(user2d, item2d, user_mlp, item_mlp)
    ueg = _sc_gather_gmf(user2d, user_gmf)
    ieg = _sc_gather_gmf(item2d, item_gmf)
    W1a, W1b = W1[:MLP], W1[MLP:]
    Wpg, Wph = Wp[:EMB], Wp[EMB:]
    return _tc_mlp(ueg, ieg, uem, iem, W1a, W1b, b1, W2, b2, W3, b3,
                   Wpg, Wph, bp.reshape(1, 1))
